# baseline (speedup 1.0000x reference)
import functools

import jax
import jax.numpy as jnp
from jax import lax
from jax.experimental import pallas as pl
from jax.experimental.pallas import tpu as pltpu
from jax.experimental.pallas import tpu_sc as plsc

N = 100000
D = 128
NC = 2
NS = 16
NW = NC * NS
CHUNK = 80
NB = CHUNK // 16
NCHUNKS = N // CHUNK
NITER = (NCHUNKS + NW - 1) // NW


def _sc_body(y_hbm, dst_hbm, emb_hbm, pout,
             idx_v, ei_v, ej_v, yi_v, yj_v, acc_s, acc_z, acc_d, stage,
             idx_sem, in_sem):
    wid = lax.axis_index("s") * NC + lax.axis_index("c")
    zero16 = jnp.zeros((16,), jnp.float32)
    acc_s[...] = zero16
    acc_z[...] = zero16
    acc_d[...] = zero16
    lane = jnp.arange(16, dtype=jnp.int32)

    def fetch_idx(t, q):
        c = wid + t * NW

        @pl.when(c < NCHUNKS)
        def _():
            pltpu.async_copy(dst_hbm.at[pl.ds(c * CHUNK, CHUNK)], idx_v[q],
                             idx_sem[q])

    def issue(t, q, p):
        c = wid + t * NW

        @pl.when(c < NCHUNKS)
        def _():
            base = c * CHUNK
            pltpu.make_async_copy(dst_hbm.at[pl.ds(0, CHUNK)], idx_v[q],
                                  idx_sem[q]).wait()
            pltpu.async_copy(emb_hbm.at[idx_v[q]], ej_v[p], in_sem[p])
            pltpu.async_copy(y_hbm.at[idx_v[q]], yj_v[p], in_sem[p])
            pltpu.async_copy(emb_hbm.at[pl.ds(base, CHUNK)], ei_v[p],
                             in_sem[p])
            pltpu.async_copy(y_hbm.at[pl.ds(base, CHUNK)], yi_v[p], in_sem[p])

    def wait_inputs(p):
        pltpu.make_async_copy(emb_hbm.at[pl.ds(0, CHUNK)], ej_v[p],
                              in_sem[p]).wait()
        pltpu.make_async_copy(y_hbm.at[pl.ds(0, CHUNK)], yj_v[p],
                              in_sem[p]).wait()
        pltpu.make_async_copy(emb_hbm.at[pl.ds(0, CHUNK)], ei_v[p],
                              in_sem[p]).wait()
        pltpu.make_async_copy(y_hbm.at[pl.ds(0, CHUNK)], yi_v[p],
                              in_sem[p]).wait()

    def compute(t, p):
        c = wid + t * NW

        @pl.when(c < NCHUNKS)
        def _():
            wait_inputs(p)
            cs = cz = cd = zero16
            for b in range(NB):
                rowi = lane + (b * 16)

                def kstep(k, car):
                    accd, accz = car
                    col = (lane + k) & (D - 1)
                    vi = plsc.load_gather(ei_v[p], [rowi, col])
                    vj = plsc.load_gather(ej_v[p], [rowi, col])
                    dl = vi - vj
                    return accd + dl * dl, accz + vi * vi

                accd, accz = lax.fori_loop(0, D, kstep, (zero16, zero16),
                                           unroll=2)

                yi_b = yi_v[p][pl.ds(b * 16, 16)]
                yj_b = yj_v[p][pl.ds(b * 16, 16)]
                same = yi_b == yj_b
                cs = cs + jnp.where(same, accd, 0.0)
                cz = cz + jnp.where(yi_b == 0, accz, 0.0)
                m = jnp.where(same, 0.0,
                              jnp.abs(yi_b - yj_b).astype(jnp.float32))
                d1 = accd + 1e-6
                bits = lax.bitcast_convert_type(d1, jnp.int32)
                rs = lax.bitcast_convert_type(
                    jnp.int32(0x5F3759DF) - (bits >> 1), jnp.float32)
                half = d1 * 0.5
                for _ in range(3):
                    rs = rs * (1.5 - half * rs * rs)
                s = d1 * rs
                tt = jnp.maximum(m - s, 0.0)
                cd = cd + tt * tt

            acc_s[...] = acc_s[...] + cs
            acc_z[...] = acc_z[...] + cz
            acc_d[...] = acc_d[...] + cd

    fetch_idx(0, 0)
    fetch_idx(1, 1)
    issue(0, 0, 0)

    def quad_body(t4, carry):
        for u in range(4):
            t = t4 + u
            p = u % 2
            issue(t + 1, (u + 1) % 4, 1 - p)
            fetch_idx(t + 2, (u + 2) % 4)
            compute(t, p)
        return carry

    lax.fori_loop(0, NITER // 4, lambda i, cr: quad_body(i * 4, cr), 0)

    stage[0, :] = acc_s[...]
    stage[1, :] = acc_z[...]
    stage[2, :] = acc_d[...]
    pltpu.sync_copy(stage, pout.at[wid])


_sc_kernel = functools.partial(
    pl.kernel,
    compiler_params=pltpu.CompilerParams(needs_layout_passes=False),
    out_type=jax.ShapeDtypeStruct((NW, 3, 16), jnp.float32),
    mesh=plsc.VectorSubcoreMesh(core_axis_name="c", subcore_axis_name="s",
                                num_cores=NC, num_subcores=NS),
    scratch_types=(
        [pltpu.VMEM((CHUNK,), jnp.int32)] * 4,
        [pltpu.VMEM((CHUNK, D), jnp.float32)] * 2,
        [pltpu.VMEM((CHUNK, D), jnp.float32)] * 2,
        [pltpu.VMEM((CHUNK,), jnp.int32)] * 2,
        [pltpu.VMEM((CHUNK,), jnp.int32)] * 2,
        pltpu.VMEM((16,), jnp.float32),
        pltpu.VMEM((16,), jnp.float32),
        pltpu.VMEM((16,), jnp.float32),
        pltpu.VMEM((3, 16), jnp.float32),
        [pltpu.SemaphoreType.DMA] * 4,
        [pltpu.SemaphoreType.DMA] * 2,
    ),
)(_sc_body)


def kernel(y_true, embeddings, src, dst):
    y = y_true.astype(jnp.int32)
    dsti = dst.astype(jnp.int32)
    partials = _sc_kernel(y, dsti, embeddings)
    return jnp.sum(partials) / jnp.float32(N)


# speedup vs baseline: 1.1240x; 1.1240x over previous
import functools

import jax
import jax.numpy as jnp
from jax import lax
from jax.experimental import pallas as pl
from jax.experimental.pallas import tpu as pltpu
from jax.experimental.pallas import tpu_sc as plsc

N = 100000
D = 128
NC = 2
NS = 16
NW = NC * NS
CHUNK = 80
NB = CHUNK // 16
NCHUNKS = N // CHUNK
NITER = (NCHUNKS + NW - 1) // NW
NDATA = 3
NIDX = 6
NSLOTS = 42


def _sc_body(y_hbm, dst_hbm, emb_hbm, pout,
             idx_v, ei_v, ej_v, yi_v, yj_v, acc_s, acc_z, acc_d, stage,
             idx_sem, in_sem):
    wid = lax.axis_index("s") * NC + lax.axis_index("c")
    zero16 = jnp.zeros((16,), jnp.float32)
    acc_s[...] = zero16
    acc_z[...] = zero16
    acc_d[...] = zero16
    lane = jnp.arange(16, dtype=jnp.int32)

    def fetch_idx(t, q):
        c = wid + t * NW

        @pl.when(c < NCHUNKS)
        def _():
            pltpu.async_copy(dst_hbm.at[pl.ds(c * CHUNK, CHUNK)], idx_v[q],
                             idx_sem[q])

    def issue(t, q, p):
        c = wid + t * NW

        @pl.when(c < NCHUNKS)
        def _():
            base = c * CHUNK
            pltpu.make_async_copy(dst_hbm.at[pl.ds(0, CHUNK)], idx_v[q],
                                  idx_sem[q]).wait()
            pltpu.async_copy(emb_hbm.at[idx_v[q]], ej_v[p], in_sem[p])
            pltpu.async_copy(y_hbm.at[idx_v[q]], yj_v[p], in_sem[p])
            pltpu.async_copy(emb_hbm.at[pl.ds(base, CHUNK)], ei_v[p],
                             in_sem[p])
            pltpu.async_copy(y_hbm.at[pl.ds(base, CHUNK)], yi_v[p], in_sem[p])

    def wait_inputs(p):
        pltpu.make_async_copy(emb_hbm.at[pl.ds(0, CHUNK)], ej_v[p],
                              in_sem[p]).wait()
        pltpu.make_async_copy(y_hbm.at[pl.ds(0, CHUNK)], yj_v[p],
                              in_sem[p]).wait()
        pltpu.make_async_copy(emb_hbm.at[pl.ds(0, CHUNK)], ei_v[p],
                              in_sem[p]).wait()
        pltpu.make_async_copy(y_hbm.at[pl.ds(0, CHUNK)], yi_v[p],
                              in_sem[p]).wait()

    def compute(t, p):
        c = wid + t * NW

        @pl.when(c < NCHUNKS)
        def _():
            wait_inputs(p)
            cs = cz = cd = zero16
            for b in range(NB):
                rowi = lane + (b * 16)

                def kstep(k4, car):
                    accs = list(car)
                    base = k4 * 4
                    for u in range(4):
                        col = (lane + (base + u)) & (D - 1)
                        vi = plsc.load_gather(ei_v[p], [rowi, col])
                        vj = plsc.load_gather(ej_v[p], [rowi, col])
                        dl = vi - vj
                        accs[u] = accs[u] + dl * dl
                        accs[4 + u] = accs[4 + u] + vi * vi
                    return tuple(accs)

                accs = plsc.parallel_loop(0, D // 4, carry=(zero16,) * 8,
                                          unroll=2)(
                    lambda k4, car: kstep(k4, car))
                accd = (accs[0] + accs[1]) + (accs[2] + accs[3])
                accz = (accs[4] + accs[5]) + (accs[6] + accs[7])

                yi_b = yi_v[p][pl.ds(b * 16, 16)]
                yj_b = yj_v[p][pl.ds(b * 16, 16)]
                same = yi_b == yj_b
                cs = cs + jnp.where(same, accd, 0.0)
                cz = cz + jnp.where(yi_b == 0, accz, 0.0)
                m = jnp.where(same, 0.0,
                              jnp.abs(yi_b - yj_b).astype(jnp.float32))
                d1 = accd + 1e-6
                bits = lax.bitcast_convert_type(d1, jnp.int32)
                rs = lax.bitcast_convert_type(
                    jnp.int32(0x5F3759DF) - (bits >> 1), jnp.float32)
                half = d1 * 0.5
                for _ in range(3):
                    rs = rs * (1.5 - half * rs * rs)
                s = d1 * rs
                tt = jnp.maximum(m - s, 0.0)
                cd = cd + tt * tt

            acc_s[...] = acc_s[...] + cs
            acc_z[...] = acc_z[...] + cz
            acc_d[...] = acc_d[...] + cd

    for t in range(4):
        fetch_idx(t, t % NIDX)
    issue(0, 0, 0)
    issue(1, 1, 1)

    def hex_body(t6, carry):
        for u in range(6):
            t = t6 + u
            issue(t + 2, (u + 2) % NIDX, (u + 2) % NDATA)
            fetch_idx(t + 4, (u + 4) % NIDX)
            compute(t, u % NDATA)
        return carry

    lax.fori_loop(0, NSLOTS // 6, lambda i, cr: hex_body(i * 6, cr), 0)

    stage[0, :] = acc_s[...]
    stage[1, :] = acc_z[...]
    stage[2, :] = acc_d[...]
    pltpu.sync_copy(stage, pout.at[wid])


_sc_kernel = functools.partial(
    pl.kernel,
    compiler_params=pltpu.CompilerParams(needs_layout_passes=False),
    out_type=jax.ShapeDtypeStruct((NW, 3, 16), jnp.float32),
    mesh=plsc.VectorSubcoreMesh(core_axis_name="c", subcore_axis_name="s",
                                num_cores=NC, num_subcores=NS),
    scratch_types=(
        [pltpu.VMEM((CHUNK,), jnp.int32)] * NIDX,
        [pltpu.VMEM((CHUNK, D), jnp.float32)] * NDATA,
        [pltpu.VMEM((CHUNK, D), jnp.float32)] * NDATA,
        [pltpu.VMEM((CHUNK,), jnp.int32)] * NDATA,
        [pltpu.VMEM((CHUNK,), jnp.int32)] * NDATA,
        pltpu.VMEM((16,), jnp.float32),
        pltpu.VMEM((16,), jnp.float32),
        pltpu.VMEM((16,), jnp.float32),
        pltpu.VMEM((3, 16), jnp.float32),
        [pltpu.SemaphoreType.DMA] * NIDX,
        [pltpu.SemaphoreType.DMA] * NDATA,
    ),
)(_sc_body)


def kernel(y_true, embeddings, src, dst):
    y = y_true.astype(jnp.int32)
    dsti = dst.astype(jnp.int32)
    partials = _sc_kernel(y, dsti, embeddings)
    return jnp.sum(partials) / jnp.float32(N)
